# baseline (speedup 1.0000x reference)
import functools

import jax
import jax.numpy as jnp
from jax import lax
from jax.experimental import pallas as pl
from jax.experimental.pallas import tpu as pltpu
from jax.experimental.pallas import tpu_sc as plsc

N = 10000
D = 128
E = 320000
NP = 10240
NW = 32
BLK = 9984
TAIL = E - NW * BLK
EB = BLK + TAIL
L = 16

_HI = jax.lax.Precision.HIGHEST
_SC_PARAMS = pltpu.CompilerParams(
    needs_layout_passes=False, disable_bounds_checks=True)


def _sc_mesh():
    return plsc.VectorSubcoreMesh(core_axis_name="c", subcore_axis_name="s")


@functools.partial(
    pl.kernel,
    out_type=jax.ShapeDtypeStruct((NW, NP), jnp.float32),
    mesh=_sc_mesh(),
    compiler_params=_SC_PARAMS,
    scratch_types=[
        pltpu.VMEM((2, EB), jnp.int32),
        pltpu.VMEM((NP,), jnp.float32),
        pltpu.SemaphoreType.DMA,
    ],
)
def _deg_kernel(ei_hbm, degp_hbm, e_v, deg_v, sem):
    wid = lax.axis_index("s") * 2 + lax.axis_index("c")
    last = wid == NW - 1
    c1 = pltpu.async_copy(
        ei_hbm.at[:, pl.ds(wid * BLK, BLK)], e_v.at[:, pl.ds(0, BLK)], sem)
    c2 = pltpu.async_copy(
        ei_hbm.at[:, pl.ds(NW * BLK, TAIL)], e_v.at[:, pl.ds(BLK, TAIL)], sem)

    zeros = jnp.zeros((L,), jnp.float32)

    @plsc.parallel_loop(0, NP // L, unroll=8)
    def _zero(i):
        deg_v[pl.ds(i * L, L)] = zeros

    c1.wait()
    c2.wait()

    ones = jnp.ones((L,), jnp.float32)

    @plsc.parallel_loop(0, BLK // L, unroll=8)
    def _edge(i):
        idx = e_v[1, pl.ds(i * L, L)]
        plsc.addupdate_scatter(deg_v, [idx], ones)

    @pl.when(last)
    def _():
        @plsc.parallel_loop(BLK // L, EB // L, unroll=8)
        def _tail(i):
            idx = e_v[1, pl.ds(i * L, L)]
            plsc.addupdate_scatter(deg_v, [idx], ones)

    pltpu.sync_copy(deg_v, degp_hbm.at[wid])


def _make_agg(F):
    @functools.partial(
        pl.kernel,
        out_type=jax.ShapeDtypeStruct((NW, 1, F * NP), jnp.float32),
        mesh=_sc_mesh(),
        compiler_params=_SC_PARAMS,
        scratch_types=[
            pltpu.VMEM((2, EB), jnp.int32),
            pltpu.VMEM((F * NP,), jnp.float32),
            pltpu.VMEM((F * NP,), jnp.float32),
            pltpu.SemaphoreType.DMA,
        ],
    )
    def _agg(g_hbm, ei_hbm, out_hbm, e_v, g_v, acc_v, sem):
        wid = lax.axis_index("s") * 2 + lax.axis_index("c")
        last = wid == NW - 1
        c1 = pltpu.async_copy(
            ei_hbm.at[:, pl.ds(wid * BLK, BLK)], e_v.at[:, pl.ds(0, BLK)], sem)
        c2 = pltpu.async_copy(
            ei_hbm.at[:, pl.ds(NW * BLK, TAIL)], e_v.at[:, pl.ds(BLK, TAIL)],
            sem)
        c3 = pltpu.async_copy(g_hbm, g_v, sem)

        zeros = jnp.zeros((L,), jnp.float32)

        @plsc.parallel_loop(0, F * NP // L, unroll=8)
        def _zero(i):
            acc_v[pl.ds(i * L, L)] = zeros

        c1.wait()
        c2.wait()
        c3.wait()

        def _body(i):
            s = e_v[0, pl.ds(i * L, L)]
            d = e_v[1, pl.ds(i * L, L)]
            for j in range(F):
                v = plsc.load_gather(g_v, [s + (j * NP)])
                plsc.addupdate_scatter(acc_v, [d + (j * NP)], v)

        @plsc.parallel_loop(0, BLK // L, unroll=8)
        def _edge(i):
            _body(i)

        @pl.when(last)
        def _():
            @plsc.parallel_loop(BLK // L, EB // L, unroll=8)
            def _tail(i):
                _body(i)

        pltpu.sync_copy(acc_v, out_hbm.at[wid, 0])

    return _agg


_agg4 = _make_agg(4)
_agg2 = _make_agg(2)
_agg1 = _make_agg(1)


def _matmul1(x, W1):
    F = W1.shape[1]

    def body(x_ref, w_ref, h_ref):
        h = lax.dot_general(w_ref[...], x_ref[...],
                            (((0,), (1,)), ((), ())), precision=_HI)
        for j in range(F):
            h_ref[pl.ds(j * NP, N)] = h[j, :]

    return pl.pallas_call(
        body,
        out_shape=jax.ShapeDtypeStruct((F * NP,), jnp.float32),
    )(x, W1)


def _prep1(degp, h1):
    F = h1.shape[0] // NP

    def body(degp_ref, h_ref, g_ref, dinv_ref):
        deg = jnp.sum(degp_ref[...], axis=0, keepdims=True) + 1.0
        dinv = 1.0 / jnp.sqrt(deg)
        dinv_ref[...] = dinv
        for j in range(F):
            blk = h_ref[pl.ds(j * NP, NP)]
            g_ref[pl.ds(j * NP, NP)] = blk * dinv[0]

    return pl.pallas_call(
        body,
        out_shape=[
            jax.ShapeDtypeStruct((F * NP,), jnp.float32),
            jax.ShapeDtypeStruct((1, NP), jnp.float32),
        ],
    )(degp, h1)


def _prep_mid(p, g, dinv, b, W):
    F, F2 = W.shape

    def body(p_ref, g_ref, dinv_ref, b_ref, w_ref, out_ref):
        psum = jnp.sum(p_ref[...], axis=0)
        dv = dinv_ref[...][0]
        o = []
        for j in range(F):
            s = psum[0, j * NP:(j + 1) * NP] + g_ref[pl.ds(j * NP, NP)]
            o.append(jnp.maximum(dv * s + b_ref[j, 0], 0.0))
        for k in range(F2):
            h = o[0] * w_ref[0, k]
            for j in range(1, F):
                h = h + o[j] * w_ref[j, k]
            out_ref[pl.ds(k * NP, NP)] = h * dv

    return pl.pallas_call(
        body,
        out_shape=jax.ShapeDtypeStruct((F2 * NP,), jnp.float32),
    )(p, g, dinv, b, W)


def _final(p, g, dinv, b):
    def body(p_ref, g_ref, dinv_ref, b_ref, out_ref):
        psum = jnp.sum(p_ref[...], axis=0)
        dv = dinv_ref[...][0]
        s = psum[0, :] + g_ref[...]
        out_ref[...] = jax.nn.sigmoid(dv * s + b_ref[0, 0])

    return pl.pallas_call(
        body,
        out_shape=jax.ShapeDtypeStruct((NP,), jnp.float32),
    )(p, g, dinv, b)


def kernel(x, edge_index, W1, b1, W2, b2, W3, b3):
    b1c = jnp.reshape(b1, (-1, 1))
    b2c = jnp.reshape(b2, (-1, 1))
    b3c = jnp.reshape(b3, (-1, 1))

    degp = _deg_kernel(edge_index)
    h1 = _matmul1(x, W1)
    g1, dinv = _prep1(degp, h1)
    p1 = _agg4(g1, edge_index)
    g2 = _prep_mid(p1, g1, dinv, b1c, W2)
    p2 = _agg2(g2, edge_index)
    g3 = _prep_mid(p2, g2, dinv, b2c, W3)
    p3 = _agg1(g3, edge_index)
    out = _final(p3, g3, dinv, b3c)
    return out[:N][:, None]


# speedup vs baseline: 1.0565x; 1.0565x over previous
import functools

import jax
import jax.numpy as jnp
from jax import lax
from jax.experimental import pallas as pl
from jax.experimental.pallas import tpu as pltpu
from jax.experimental.pallas import tpu_sc as plsc

N = 10000
D = 128
E = 320000
NP = 10240
NW = 32
BLK = 9984
TAIL = E - NW * BLK
EB = BLK + TAIL
L = 16

_HI = jax.lax.Precision.HIGHEST
_SC_PARAMS = pltpu.CompilerParams(
    needs_layout_passes=False, disable_bounds_checks=True)


def _sc_mesh():
    return plsc.VectorSubcoreMesh(core_axis_name="c", subcore_axis_name="s")


@functools.partial(
    pl.kernel,
    out_type=jax.ShapeDtypeStruct((NW, NP), jnp.float32),
    mesh=_sc_mesh(),
    compiler_params=_SC_PARAMS,
    scratch_types=[
        pltpu.VMEM((2, EB), jnp.int32),
        pltpu.VMEM((NP,), jnp.float32),
        pltpu.SemaphoreType.DMA,
    ],
)
def _deg_kernel(ei_hbm, degp_hbm, e_v, deg_v, sem):
    wid = lax.axis_index("s") * 2 + lax.axis_index("c")
    last = wid == NW - 1
    c1 = pltpu.async_copy(
        ei_hbm.at[:, pl.ds(wid * BLK, BLK)], e_v.at[:, pl.ds(0, BLK)], sem)
    c2 = pltpu.async_copy(
        ei_hbm.at[:, pl.ds(NW * BLK, TAIL)], e_v.at[:, pl.ds(BLK, TAIL)], sem)

    zeros = jnp.zeros((L,), jnp.float32)

    @plsc.parallel_loop(0, NP // L, unroll=8)
    def _zero(i):
        deg_v[pl.ds(i * L, L)] = zeros

    c1.wait()
    c2.wait()

    ones = jnp.ones((L,), jnp.float32)

    @plsc.parallel_loop(0, BLK // L, unroll=8)
    def _edge(i):
        idx = e_v[1, pl.ds(i * L, L)]
        plsc.addupdate_scatter(deg_v, [idx], ones)

    @pl.when(last)
    def _():
        @plsc.parallel_loop(BLK // L, EB // L, unroll=8)
        def _tail(i):
            idx = e_v[1, pl.ds(i * L, L)]
            plsc.addupdate_scatter(deg_v, [idx], ones)

    pltpu.sync_copy(deg_v, degp_hbm.at[wid])


def _make_agg(F):
    @functools.partial(
        pl.kernel,
        out_type=jax.ShapeDtypeStruct((NW, F, NP), jnp.float32),
        mesh=_sc_mesh(),
        compiler_params=_SC_PARAMS,
        scratch_types=[
            pltpu.VMEM((2, EB), jnp.int32),
            pltpu.VMEM((F, NP), jnp.float32),
            pltpu.VMEM((F, NP), jnp.float32),
            pltpu.SemaphoreType.DMA,
        ],
    )
    def _agg(g_hbm, ei_hbm, out_hbm, e_v, g_v, acc_v, sem):
        wid = lax.axis_index("s") * 2 + lax.axis_index("c")
        last = wid == NW - 1
        c1 = pltpu.async_copy(
            ei_hbm.at[:, pl.ds(wid * BLK, BLK)], e_v.at[:, pl.ds(0, BLK)], sem)
        c2 = pltpu.async_copy(
            ei_hbm.at[:, pl.ds(NW * BLK, TAIL)], e_v.at[:, pl.ds(BLK, TAIL)],
            sem)
        c3 = pltpu.async_copy(g_hbm, g_v, sem)

        zeros = jnp.zeros((L,), jnp.float32)

        @plsc.parallel_loop(0, NP // L, unroll=8)
        def _zero(i):
            for j in range(F):
                acc_v[j, pl.ds(i * L, L)] = zeros

        c1.wait()
        c2.wait()
        c3.wait()

        def _body(i):
            s = e_v[0, pl.ds(i * L, L)]
            d = e_v[1, pl.ds(i * L, L)]
            for j in range(F):
                jv = jnp.full((L,), j, jnp.int32)
                v = plsc.load_gather(g_v, [jv, s])
                plsc.addupdate_scatter(acc_v, [jv, d], v)

        @plsc.parallel_loop(0, BLK // L, unroll=16)
        def _edge(i):
            _body(i)

        @pl.when(last)
        def _():
            @plsc.parallel_loop(BLK // L, EB // L, unroll=16)
            def _tail(i):
                _body(i)

        pltpu.sync_copy(acc_v, out_hbm.at[wid])

    return _agg


_agg4 = _make_agg(4)
_agg2 = _make_agg(2)
_agg1 = _make_agg(1)


def _matmul1(x, W1):
    def body(x_ref, w_ref, h_ref):
        h = lax.dot_general(w_ref[...], x_ref[...],
                            (((0,), (1,)), ((), ())), precision=_HI)
        h_ref[:, pl.ds(0, N)] = h

    return pl.pallas_call(
        body,
        out_shape=jax.ShapeDtypeStruct((W1.shape[1], NP), jnp.float32),
    )(x, W1)


def _prep1(degp, h1):
    def body(degp_ref, h_ref, g_ref, dinv_ref):
        deg = jnp.sum(degp_ref[...], axis=0, keepdims=True) + 1.0
        dinv = 1.0 / jnp.sqrt(deg)
        g_ref[...] = h_ref[...] * dinv
        dinv_ref[...] = dinv

    return pl.pallas_call(
        body,
        out_shape=[
            jax.ShapeDtypeStruct(h1.shape, jnp.float32),
            jax.ShapeDtypeStruct((1, NP), jnp.float32),
        ],
    )(degp, h1)


def _prep_mid(p, g, dinv, b, W):
    def body(p_ref, g_ref, dinv_ref, b_ref, w_ref, out_ref):
        s = g_ref[...] + p_ref[0]
        for k in range(1, NW):
            s = s + p_ref[k]
        o = jnp.maximum(dinv_ref[...] * s + b_ref[...], 0.0)
        h = lax.dot_general(w_ref[...], o,
                            (((0,), (0,)), ((), ())), precision=_HI)
        out_ref[...] = h * dinv_ref[...]

    return pl.pallas_call(
        body,
        out_shape=jax.ShapeDtypeStruct((W.shape[1], NP), jnp.float32),
    )(p, g, dinv, b, W)


def _final(p, g, dinv, b):
    def body(p_ref, g_ref, dinv_ref, b_ref, out_ref):
        s = g_ref[...] + p_ref[0]
        for k in range(1, NW):
            s = s + p_ref[k]
        out_ref[...] = jax.nn.sigmoid(dinv_ref[...] * s + b_ref[...])

    return pl.pallas_call(
        body,
        out_shape=jax.ShapeDtypeStruct((1, NP), jnp.float32),
    )(p, g, dinv, b)


def kernel(x, edge_index, W1, b1, W2, b2, W3, b3):
    b1c = jnp.reshape(b1, (-1, 1))
    b2c = jnp.reshape(b2, (-1, 1))
    b3c = jnp.reshape(b3, (-1, 1))

    degp = _deg_kernel(edge_index)
    h1 = _matmul1(x, W1)
    g1, dinv = _prep1(degp, h1)
    p1 = _agg4(g1, edge_index)
    g2 = _prep_mid(p1, g1, dinv, b1c, W2)
    p2 = _agg2(g2, edge_index)
    g3 = _prep_mid(p2, g2, dinv, b2c, W3)
    p3 = _agg1(g3, edge_index)
    out = _final(p3, g3, dinv, b3c)
    return out[0, :N][:, None]
